# Initial kernel scaffold; baseline (speedup 1.0000x reference)
import jax
import jax.numpy as jnp
from jax.experimental import pallas as pl


def kernel(x, in_proj_w, in_proj_b, out_proj_w, out_proj_b, ln1_w, ln1_b, ln2_w, ln2_b, router_w, e_bias, gate_w, up_w, down_w, sh_gate_w, sh_up_w, sh_down_w):
    raise NotImplementedError("write your pallas kernel here")


# speedup vs baseline: 1.5278x; 1.5278x over previous
import functools
import math

import jax
import jax.numpy as jnp
from jax import lax
from jax.experimental import pallas as pl
from jax.experimental.pallas import tpu as pltpu
from jax.experimental.pallas import tpu_sc as plsc

D = 2048
H = 16
DH = D // H
M = 1024
E = 8
K = 2
NG = 4
T = 2048
NA = T * K
G = 256
NPAD = NA + E * G
NBLK = NPAD // G

_HIGH = lax.Precision.HIGHEST
f32 = jnp.float32
bf16 = jnp.bfloat16


def _dot_nt(a, b, precision=None):
    return lax.dot_general(a, b, (((1,), (1,)), ((), ())),
                           preferred_element_type=f32, precision=precision)


def _dot_nn(a, b, precision=None):
    return lax.dot_general(a, b, (((1,), (0,)), ((), ())),
                           preferred_element_type=f32, precision=precision)


def _qkv_body(x_ref, w_ref, b_ref, o_ref):
    acc = _dot_nt(x_ref[...], w_ref[...])
    o_ref[...] = (acc + b_ref[...]).astype(bf16)


def _qkv_proj(xbf, wbf, b2d):
    return pl.pallas_call(
        _qkv_body,
        grid=(12,),
        in_specs=[
            pl.BlockSpec((T, D), lambda n: (0, 0)),
            pl.BlockSpec((512, D), lambda n: (n, 0)),
            pl.BlockSpec((1, 512), lambda n: (0, n)),
        ],
        out_specs=pl.BlockSpec((T, 512), lambda n: (0, n)),
        out_shape=jax.ShapeDtypeStruct((T, 3 * D), bf16),
    )(xbf, wbf, b2d)


def _attn_body(q_ref, k_ref, v_ref, o_ref):
    q = q_ref[...]
    k = k_ref[...]
    s = _dot_nt(q, k) / jnp.sqrt(jnp.float32(DH))
    m = jnp.max(s, axis=1, keepdims=True)
    p = jnp.exp(s - m)
    att = p / jnp.sum(p, axis=1, keepdims=True)
    o_ref[...] = _dot_nn(att.astype(bf16), v_ref[...]).astype(bf16)


def _attention(qkvbf):
    return pl.pallas_call(
        _attn_body,
        grid=(H,),
        in_specs=[
            pl.BlockSpec((T, DH), lambda h: (0, h)),
            pl.BlockSpec((T, DH), lambda h: (0, 16 + h)),
            pl.BlockSpec((T, DH), lambda h: (0, 32 + h)),
        ],
        out_specs=pl.BlockSpec((T, DH), lambda h: (0, h)),
        out_shape=jax.ShapeDtypeStruct((T, D), bf16),
    )(qkvbf, qkvbf, qkvbf)


def _postattn_body(ao_ref, w_ref, b_ref, x_ref, lw_ref, lb_ref, rw_ref,
                   x1_ref, lg_ref):
    ao = _dot_nt(ao_ref[...], w_ref[...]) + b_ref[...]
    y = x_ref[...] + ao
    mu = jnp.mean(y, axis=1, keepdims=True)
    var = jnp.mean((y - mu) ** 2, axis=1, keepdims=True)
    x1 = (y - mu) / jnp.sqrt(var + 1e-5) * lw_ref[...] + lb_ref[...]
    x1_ref[...] = x1
    lg_ref[...] = _dot_nt(x1.astype(bf16), rw_ref[...])


def _postattn(aobf, woutbf, outb2d, x2d, ln1w2d, ln1b2d, rwpad):
    return pl.pallas_call(
        _postattn_body,
        grid=(8,),
        in_specs=[
            pl.BlockSpec((256, D), lambda r: (r, 0)),
            pl.BlockSpec((D, D), lambda r: (0, 0)),
            pl.BlockSpec((1, D), lambda r: (0, 0)),
            pl.BlockSpec((256, D), lambda r: (r, 0)),
            pl.BlockSpec((1, D), lambda r: (0, 0)),
            pl.BlockSpec((1, D), lambda r: (0, 0)),
            pl.BlockSpec((128, D), lambda r: (0, 0)),
        ],
        out_specs=[
            pl.BlockSpec((256, D), lambda r: (r, 0)),
            pl.BlockSpec((256, 128), lambda r: (r, 0)),
        ],
        out_shape=[
            jax.ShapeDtypeStruct((T, D), f32),
            jax.ShapeDtypeStruct((T, 128), f32),
        ],
    )(aobf, woutbf, outb2d, x2d, ln1w2d, ln1b2d, rwpad)


def _iotaf(shape, dim):
    return lax.broadcasted_iota(jnp.int32, shape, dim).astype(f32)


def _argmax_first(vals, lane):
    m = jnp.max(vals, axis=1, keepdims=True)
    hit = vals == m
    return jnp.min(jnp.where(hit, lane, 1e9), axis=1, keepdims=True)


def _route_body(lg_ref, eb_ref, wp_ref, pos_ref, sched_ref, valid_ref):
    lane = _iotaf((T, 128), 1)
    lane8 = lane < 8.0
    logits = lg_ref[...]
    scores = jnp.where(lane8, jax.nn.sigmoid(logits), 0.0)
    sfc = jnp.where(lane8, scores + eb_ref[...], 0.0)

    znext = jnp.concatenate([sfc[:, 1:], jnp.zeros((T, 1), f32)], axis=1)
    zprev = jnp.concatenate([jnp.zeros((T, 1), f32), sfc[:, :-1]], axis=1)
    ilane = lax.broadcasted_iota(jnp.int32, (T, 128), 1)
    even = (ilane & 1) == 0
    gsum = sfc + jnp.where(even, znext, zprev)

    grp_vals = jnp.where(even & lane8, gsum, -1.0)
    g1 = _argmax_first(grp_vals, lane)
    grp_vals2 = jnp.where(lane == g1, -1.0, grp_vals)
    g2 = _argmax_first(grp_vals2, lane)
    gbase = (ilane - (ilane & 1)).astype(f32)
    emask = (gbase == g1) | (gbase == g2)

    masked = jnp.where(lane8, jnp.where(emask, sfc, 0.0), -1.0)
    a1 = _argmax_first(masked, lane)
    masked2 = jnp.where(lane == a1, -2.0, masked)
    a2 = _argmax_first(masked2, lane)

    w1 = jnp.sum(jnp.where(lane == a1, scores, 0.0), axis=1, keepdims=True)
    w2 = jnp.sum(jnp.where(lane == a2, scores, 0.0), axis=1, keepdims=True)
    ws = w1 + w2 + 1e-20
    wp_ref[...] = jnp.where(lane == 0.0, w1 / ws,
                            jnp.where(lane == 1.0, w2 / ws, 0.0))

    r512 = _iotaf((512, 512), 0)
    c512 = _iotaf((512, 512), 1)
    tril_inc = (c512 <= r512).astype(f32)
    lane512 = _iotaf((512, 128), 1)

    def chunk_expert(c):
        if c < 4:
            return a1[c * 512:(c + 1) * 512, :]
        return a2[(c - 4) * 512:(c - 3) * 512, :]

    tots = []
    for c in range(8):
        a_c = (lane512 == chunk_expert(c)).astype(f32)
        rci = _dot_nn(tril_inc, a_c, precision=_HIGH)
        tots.append(rci[511:512, :])
    tots8 = jnp.concatenate(tots, axis=0)
    counts = jnp.sum(tots8, axis=0, keepdims=True)

    r8 = _iotaf((8, 8), 0)
    c8 = _iotaf((8, 8), 1)
    tril8s = (c8 < r8).astype(f32)
    chunk_pre = _dot_nn(tril8s, tots8, precision=_HIGH)

    pc = jnp.ceil(counts / G) * G
    r128 = _iotaf((128, 128), 0)
    cc128 = _iotaf((128, 128), 1)
    upper_s = (r128 < cc128).astype(f32)
    seg_start = _dot_nn(pc, upper_s, precision=_HIGH)

    for c in range(8):
        a_c = (lane512 == chunk_expert(c)).astype(f32)
        rci = _dot_nn(tril_inc, a_c, precision=_HIGH)
        rce = rci - a_c
        pre = chunk_pre[c:c + 1, :]
        pos_dense = rce + pre + seg_start
        posc = jnp.sum(a_c * pos_dense, axis=1, keepdims=True)
        pos_ref[pl.ds(c * 512, 512), :] = posc.astype(jnp.int32)

    nb = pc / G
    sb = seg_start / G
    brow = _iotaf((32, 128), 0)
    lane32 = _iotaf((32, 128), 1)
    mat = ((brow >= sb) & (brow < sb + nb) & (lane32 < 8.0)).astype(f32)
    be = jnp.sum(mat * lane32, axis=1, keepdims=True)
    vld = jnp.sum(mat, axis=1, keepdims=True)
    last_e = jnp.max(jnp.where((counts > 0) & (lane32[0:1] < 8.0),
                               lane32[0:1], -1.0))
    sched_ref[...] = jnp.where(vld > 0, be, last_e).astype(jnp.int32)
    valid_ref[...] = vld.astype(jnp.int32)


def _route(logits, ebpad):
    return pl.pallas_call(
        _route_body,
        grid=(1,),
        in_specs=[
            pl.BlockSpec((T, 128), lambda i: (0, 0)),
            pl.BlockSpec((1, 128), lambda i: (0, 0)),
        ],
        out_specs=[
            pl.BlockSpec((T, 128), lambda i: (0, 0)),
            pl.BlockSpec((NA, 1), lambda i: (0, 0)),
            pl.BlockSpec((32, 1), lambda i: (0, 0)),
            pl.BlockSpec((32, 1), lambda i: (0, 0)),
        ],
        out_shape=[
            jax.ShapeDtypeStruct((T, 128), f32),
            jax.ShapeDtypeStruct((NA, 1), jnp.int32),
            jax.ShapeDtypeStruct((32, 1), jnp.int32),
            jax.ShapeDtypeStruct((32, 1), jnp.int32),
        ],
    )(logits, ebpad)


def _sc_mesh():
    return plsc.VectorSubcoreMesh(core_axis_name="c", subcore_axis_name="s")


_NW = 32
_CH = 32
_NCH = NA // (_NW * _CH)


def _sc_scatter_rows(x1, pos3):
    @functools.partial(
        pl.kernel,
        mesh=_sc_mesh(),
        out_type=jax.ShapeDtypeStruct((NPAD, D), f32),
        scratch_types=[
            pltpu.VMEM((_CH,), jnp.int32),
            pltpu.VMEM((_CH, D), f32),
            pltpu.SemaphoreType.DMA,
        ],
    )
    def k(x1_hbm, pos_hbm, out_hbm, idx_v, rows_v, sem):
        wid = lax.axis_index("s") * 2 + lax.axis_index("c")
        for j in range(_NCH):
            a0 = wid * (_CH * _NCH) + j * _CH
            t0 = lax.rem(a0, T)
            pltpu.sync_copy(pos_hbm.at[wid, j], idx_v)
            pltpu.sync_copy(x1_hbm.at[pl.ds(t0, _CH)], rows_v)
            pltpu.async_copy(rows_v, out_hbm.at[idx_v], sem).wait()

    return k(x1, pos3)


def _sc_gather_rows(hs, pos3):
    @functools.partial(
        pl.kernel,
        mesh=_sc_mesh(),
        out_type=jax.ShapeDtypeStruct((NA, D), f32),
        scratch_types=[
            pltpu.VMEM((_CH,), jnp.int32),
            pltpu.VMEM((_CH, D), f32),
            pltpu.SemaphoreType.DMA,
        ],
    )
    def k(h_hbm, pos_hbm, out_hbm, idx_v, rows_v, sem):
        wid = lax.axis_index("s") * 2 + lax.axis_index("c")
        for j in range(_NCH):
            base = wid * (_CH * _NCH) + j * _CH
            pltpu.sync_copy(pos_hbm.at[wid, j], idx_v)
            pltpu.async_copy(h_hbm.at[idx_v], rows_v, sem).wait()
            pltpu.sync_copy(rows_v, out_hbm.at[pl.ds(base, _CH)])

    return k(hs, pos3)


def _moe_body(sched_ref, valid_ref, xs_ref, g_ref, u_ref, d_ref, o_ref):
    i = pl.program_id(0)

    @pl.when(valid_ref[i] == 1)
    def _():
        xb = xs_ref[...].astype(bf16)
        g = _dot_nt(xb, g_ref[0])
        u = _dot_nt(xb, u_ref[0])
        act = (g * jax.nn.sigmoid(g) * u).astype(bf16)
        o_ref[...] = _dot_nt(act, d_ref[0])


def _moe_grouped(sched1d, valid1d, xs, gbf, ubf, dbf):
    grid_spec = pltpu.PrefetchScalarGridSpec(
        num_scalar_prefetch=2,
        grid=(NBLK,),
        in_specs=[
            pl.BlockSpec((G, D), lambda i, sr, vr: (i, 0)),
            pl.BlockSpec((1, M, D), lambda i, sr, vr: (sr[i], 0, 0)),
            pl.BlockSpec((1, M, D), lambda i, sr, vr: (sr[i], 0, 0)),
            pl.BlockSpec((1, D, M), lambda i, sr, vr: (sr[i], 0, 0)),
        ],
        out_specs=pl.BlockSpec((G, D), lambda i, sr, vr: (i, 0)),
    )
    return pl.pallas_call(
        _moe_body,
        grid_spec=grid_spec,
        out_shape=jax.ShapeDtypeStruct((NPAD, D), f32),
    )(sched1d, valid1d, xs, gbf, ubf, dbf)


def _shared_body(x_ref, g_ref, u_ref, d_ref, o_ref):
    xb = x_ref[...].astype(bf16)
    g = _dot_nt(xb, g_ref[...])
    u = _dot_nt(xb, u_ref[...])
    act = (g * jax.nn.sigmoid(g) * u).astype(bf16)
    o_ref[...] = _dot_nt(act, d_ref[...])


def _shared_expert(x1, shgbf, shubf, shdbf):
    return pl.pallas_call(
        _shared_body,
        grid=(4,),
        in_specs=[
            pl.BlockSpec((512, D), lambda r: (r, 0)),
            pl.BlockSpec((M, D), lambda r: (0, 0)),
            pl.BlockSpec((M, D), lambda r: (0, 0)),
            pl.BlockSpec((D, M), lambda r: (0, 0)),
        ],
        out_specs=pl.BlockSpec((512, D), lambda r: (r, 0)),
        out_shape=jax.ShapeDtypeStruct((T, D), f32),
    )(x1, shgbf, shubf, shdbf)


def _combine_body(h0_ref, h1_ref, wp_ref, sh_ref, x1_ref, lw_ref, lb_ref,
                  o_ref):
    w0 = wp_ref[:, 0:1]
    w1 = wp_ref[:, 1:2]
    y = x1_ref[...] + sh_ref[...] + w0 * h0_ref[...] + w1 * h1_ref[...]
    mu = jnp.mean(y, axis=1, keepdims=True)
    var = jnp.mean((y - mu) ** 2, axis=1, keepdims=True)
    o_ref[...] = (y - mu) / jnp.sqrt(var + 1e-5) * lw_ref[...] + lb_ref[...]


def _combine(h01, wpair, shared, x1, ln2w2d, ln2b2d):
    return pl.pallas_call(
        _combine_body,
        grid=(4,),
        in_specs=[
            pl.BlockSpec((512, D), lambda r: (r, 0)),
            pl.BlockSpec((512, D), lambda r: (4 + r, 0)),
            pl.BlockSpec((512, 128), lambda r: (r, 0)),
            pl.BlockSpec((512, D), lambda r: (r, 0)),
            pl.BlockSpec((512, D), lambda r: (r, 0)),
            pl.BlockSpec((1, D), lambda r: (0, 0)),
            pl.BlockSpec((1, D), lambda r: (0, 0)),
        ],
        out_specs=pl.BlockSpec((512, D), lambda r: (r, 0)),
        out_shape=jax.ShapeDtypeStruct((T, D), f32),
    )(h01, h01, wpair, shared, x1, ln2w2d, ln2b2d)


def kernel(x, in_proj_w, in_proj_b, out_proj_w, out_proj_b, ln1_w, ln1_b,
           ln2_w, ln2_b, router_w, e_bias, gate_w, up_w, down_w, sh_gate_w,
           sh_up_w, sh_down_w):
    x2d = x.reshape(T, D)
    xbf = x2d.astype(bf16)

    qkv = _qkv_proj(xbf, in_proj_w.astype(bf16), in_proj_b.reshape(1, 3 * D))
    ao = _attention(qkv)

    rwpad = jnp.zeros((128, D), bf16).at[:E].set(router_w.astype(bf16))
    x1, logits = _postattn(ao, out_proj_w.astype(bf16),
                           out_proj_b.reshape(1, D), x2d,
                           ln1_w.reshape(1, D), ln1_b.reshape(1, D), rwpad)

    ebpad = jnp.zeros((1, 128), f32).at[0, :E].set(e_bias)
    wpair, pos, sched, valid = _route(logits, ebpad)
    pos3 = pos.reshape(_NW, _NCH, _CH)
    sched1d = sched[:NBLK, 0]
    valid1d = valid[:NBLK, 0]

    xs = _sc_scatter_rows(x1, pos3)
    shared = _shared_expert(x1, sh_gate_w.astype(bf16), sh_up_w.astype(bf16),
                            sh_down_w.astype(bf16))

    hs = _moe_grouped(sched1d, valid1d, xs, gate_w.astype(bf16),
                      up_w.astype(bf16), down_w.astype(bf16))
    h01 = _sc_gather_rows(hs, pos3)

    out = _combine(h01, wpair, shared, x1, ln2_w.reshape(1, D),
                   ln2_b.reshape(1, D))
    return out.reshape(1, T, D)


# speedup vs baseline: 1.5301x; 1.0015x over previous
import functools
import math

import jax
import jax.numpy as jnp
from jax import lax
from jax.experimental import pallas as pl
from jax.experimental.pallas import tpu as pltpu
from jax.experimental.pallas import tpu_sc as plsc

D = 2048
H = 16
DH = D // H
M = 1024
E = 8
K = 2
NG = 4
T = 2048
NA = T * K
G = 256
NPAD = NA + E * G
NBLK = NPAD // G

_HIGH = lax.Precision.HIGHEST
f32 = jnp.float32
bf16 = jnp.bfloat16


def _dot_nt(a, b, precision=None):
    return lax.dot_general(a, b, (((1,), (1,)), ((), ())),
                           preferred_element_type=f32, precision=precision)


def _dot_nn(a, b, precision=None):
    return lax.dot_general(a, b, (((1,), (0,)), ((), ())),
                           preferred_element_type=f32, precision=precision)


def _qkv_body(x_ref, w_ref, b_ref, o_ref):
    acc = _dot_nt(x_ref[...], w_ref[...])
    o_ref[...] = (acc + b_ref[...]).astype(bf16)


def _qkv_proj(xbf, wbf, b2d):
    return pl.pallas_call(
        _qkv_body,
        grid=(12,),
        in_specs=[
            pl.BlockSpec((T, D), lambda n: (0, 0)),
            pl.BlockSpec((512, D), lambda n: (n, 0)),
            pl.BlockSpec((1, 512), lambda n: (0, n)),
        ],
        out_specs=pl.BlockSpec((T, 512), lambda n: (0, n)),
        out_shape=jax.ShapeDtypeStruct((T, 3 * D), bf16),
    )(xbf, wbf, b2d)


def _attn_body(q_ref, k_ref, v_ref, o_ref):
    q = q_ref[...]
    k = k_ref[...]
    s = _dot_nt(q, k) / jnp.sqrt(jnp.float32(DH))
    m = jnp.max(s, axis=1, keepdims=True)
    p = jnp.exp(s - m)
    att = p / jnp.sum(p, axis=1, keepdims=True)
    o_ref[...] = _dot_nn(att.astype(bf16), v_ref[...]).astype(bf16)


def _attention(qkvbf):
    return pl.pallas_call(
        _attn_body,
        grid=(H,),
        in_specs=[
            pl.BlockSpec((T, DH), lambda h: (0, h)),
            pl.BlockSpec((T, DH), lambda h: (0, 16 + h)),
            pl.BlockSpec((T, DH), lambda h: (0, 32 + h)),
        ],
        out_specs=pl.BlockSpec((T, DH), lambda h: (0, h)),
        out_shape=jax.ShapeDtypeStruct((T, D), bf16),
    )(qkvbf, qkvbf, qkvbf)


def _postattn_body(ao_ref, w_ref, b_ref, x_ref, lw_ref, lb_ref, rw_ref,
                   x1_ref, lg_ref):
    ao = _dot_nt(ao_ref[...], w_ref[...]) + b_ref[...]
    y = x_ref[...] + ao
    mu = jnp.mean(y, axis=1, keepdims=True)
    var = jnp.mean((y - mu) ** 2, axis=1, keepdims=True)
    x1 = (y - mu) / jnp.sqrt(var + 1e-5) * lw_ref[...] + lb_ref[...]
    x1_ref[...] = x1
    lg_ref[...] = _dot_nt(x1.astype(bf16), rw_ref[...])


def _postattn(aobf, woutbf, outb2d, x2d, ln1w2d, ln1b2d, rwpad):
    return pl.pallas_call(
        _postattn_body,
        grid=(8,),
        in_specs=[
            pl.BlockSpec((256, D), lambda r: (r, 0)),
            pl.BlockSpec((D, D), lambda r: (0, 0)),
            pl.BlockSpec((1, D), lambda r: (0, 0)),
            pl.BlockSpec((256, D), lambda r: (r, 0)),
            pl.BlockSpec((1, D), lambda r: (0, 0)),
            pl.BlockSpec((1, D), lambda r: (0, 0)),
            pl.BlockSpec((128, D), lambda r: (0, 0)),
        ],
        out_specs=[
            pl.BlockSpec((256, D), lambda r: (r, 0)),
            pl.BlockSpec((256, 128), lambda r: (r, 0)),
        ],
        out_shape=[
            jax.ShapeDtypeStruct((T, D), f32),
            jax.ShapeDtypeStruct((T, 128), f32),
        ],
    )(aobf, woutbf, outb2d, x2d, ln1w2d, ln1b2d, rwpad)


def _iotaf(shape, dim):
    return lax.broadcasted_iota(jnp.int32, shape, dim).astype(f32)


def _argmax_first(vals, lane):
    m = jnp.max(vals, axis=1, keepdims=True)
    hit = vals == m
    return jnp.min(jnp.where(hit, lane, 1e9), axis=1, keepdims=True)


def _route_body(lg_ref, eb_ref, wp_ref, pos_ref, sched_ref, valid_ref):
    lane = _iotaf((T, 128), 1)
    lane8 = lane < 8.0
    logits = lg_ref[...]
    scores = jnp.where(lane8, jax.nn.sigmoid(logits), 0.0)
    sfc = jnp.where(lane8, scores + eb_ref[...], 0.0)

    znext = jnp.concatenate([sfc[:, 1:], jnp.zeros((T, 1), f32)], axis=1)
    zprev = jnp.concatenate([jnp.zeros((T, 1), f32), sfc[:, :-1]], axis=1)
    ilane = lax.broadcasted_iota(jnp.int32, (T, 128), 1)
    even = (ilane & 1) == 0
    gsum = sfc + jnp.where(even, znext, zprev)

    grp_vals = jnp.where(even & lane8, gsum, -1.0)
    g1 = _argmax_first(grp_vals, lane)
    grp_vals2 = jnp.where(lane == g1, -1.0, grp_vals)
    g2 = _argmax_first(grp_vals2, lane)
    gbase = (ilane - (ilane & 1)).astype(f32)
    emask = (gbase == g1) | (gbase == g2)

    masked = jnp.where(lane8, jnp.where(emask, sfc, 0.0), -1.0)
    a1 = _argmax_first(masked, lane)
    masked2 = jnp.where(lane == a1, -2.0, masked)
    a2 = _argmax_first(masked2, lane)

    w1 = jnp.sum(jnp.where(lane == a1, scores, 0.0), axis=1, keepdims=True)
    w2 = jnp.sum(jnp.where(lane == a2, scores, 0.0), axis=1, keepdims=True)
    ws = w1 + w2 + 1e-20
    wp_ref[...] = jnp.where(lane == 0.0, w1 / ws,
                            jnp.where(lane == 1.0, w2 / ws, 0.0))

    r512 = _iotaf((512, 512), 0)
    c512 = _iotaf((512, 512), 1)
    tril_inc = (c512 <= r512).astype(f32)
    lane512 = _iotaf((512, 128), 1)

    def chunk_expert(c):
        if c < 4:
            return a1[c * 512:(c + 1) * 512, :]
        return a2[(c - 4) * 512:(c - 3) * 512, :]

    tots = []
    for c in range(8):
        a_c = (lane512 == chunk_expert(c)).astype(f32)
        rci = _dot_nn(tril_inc, a_c, precision=_HIGH)
        tots.append(rci[511:512, :])
    tots8 = jnp.concatenate(tots, axis=0)
    counts = jnp.sum(tots8, axis=0, keepdims=True)

    r8 = _iotaf((8, 8), 0)
    c8 = _iotaf((8, 8), 1)
    tril8s = (c8 < r8).astype(f32)
    chunk_pre = _dot_nn(tril8s, tots8, precision=_HIGH)

    pc = jnp.ceil(counts / G) * G
    r128 = _iotaf((128, 128), 0)
    cc128 = _iotaf((128, 128), 1)
    upper_s = (r128 < cc128).astype(f32)
    seg_start = _dot_nn(pc, upper_s, precision=_HIGH)

    for c in range(8):
        a_c = (lane512 == chunk_expert(c)).astype(f32)
        rci = _dot_nn(tril_inc, a_c, precision=_HIGH)
        rce = rci - a_c
        pre = chunk_pre[c:c + 1, :]
        pos_dense = rce + pre + seg_start
        posc = jnp.sum(a_c * pos_dense, axis=1, keepdims=True)
        pos_ref[pl.ds(c * 512, 512), :] = posc.astype(jnp.int32)

    nb = pc / G
    sb = seg_start / G
    brow = _iotaf((32, 128), 0)
    lane32 = _iotaf((32, 128), 1)
    mat = ((brow >= sb) & (brow < sb + nb) & (lane32 < 8.0)).astype(f32)
    be = jnp.sum(mat * lane32, axis=1, keepdims=True)
    vld = jnp.sum(mat, axis=1, keepdims=True)
    last_e = jnp.max(jnp.where((counts > 0) & (lane32[0:1] < 8.0),
                               lane32[0:1], -1.0))
    sched_ref[...] = jnp.where(vld > 0, be, last_e).astype(jnp.int32)
    valid_ref[...] = vld.astype(jnp.int32)


def _route(logits, ebpad):
    return pl.pallas_call(
        _route_body,
        grid=(1,),
        in_specs=[
            pl.BlockSpec((T, 128), lambda i: (0, 0)),
            pl.BlockSpec((1, 128), lambda i: (0, 0)),
        ],
        out_specs=[
            pl.BlockSpec((T, 128), lambda i: (0, 0)),
            pl.BlockSpec((NA, 1), lambda i: (0, 0)),
            pl.BlockSpec((32, 1), lambda i: (0, 0)),
            pl.BlockSpec((32, 1), lambda i: (0, 0)),
        ],
        out_shape=[
            jax.ShapeDtypeStruct((T, 128), f32),
            jax.ShapeDtypeStruct((NA, 1), jnp.int32),
            jax.ShapeDtypeStruct((32, 1), jnp.int32),
            jax.ShapeDtypeStruct((32, 1), jnp.int32),
        ],
    )(logits, ebpad)


def _sc_mesh():
    return plsc.VectorSubcoreMesh(core_axis_name="c", subcore_axis_name="s")


_NW = 32
_CH = 32
_NCH = NA // (_NW * _CH)


def _sc_scatter_rows(x1, pos3):
    @functools.partial(
        pl.kernel,
        mesh=_sc_mesh(),
        out_type=jax.ShapeDtypeStruct((NPAD, D), f32),
        scratch_types=[
            pltpu.VMEM((_CH,), jnp.int32),
            pltpu.VMEM((_CH, D), f32),
            pltpu.SemaphoreType.DMA,
        ],
    )
    def k(x1_hbm, pos_hbm, out_hbm, idx_v, rows_v, sem):
        wid = lax.axis_index("s") * 2 + lax.axis_index("c")
        for j in range(_NCH):
            a0 = wid * (_CH * _NCH) + j * _CH
            t0 = lax.rem(a0, T)
            pltpu.sync_copy(pos_hbm.at[wid, j], idx_v)
            pltpu.sync_copy(x1_hbm.at[pl.ds(t0, _CH)], rows_v)
            pltpu.async_copy(rows_v, out_hbm.at[idx_v], sem).wait()

    return k(x1, pos3)


def _sc_gather_rows(hs, pos3):
    @functools.partial(
        pl.kernel,
        mesh=_sc_mesh(),
        out_type=jax.ShapeDtypeStruct((NA, D), f32),
        scratch_types=[
            pltpu.VMEM((_CH,), jnp.int32),
            pltpu.VMEM((_CH, D), f32),
            pltpu.SemaphoreType.DMA,
        ],
    )
    def k(h_hbm, pos_hbm, out_hbm, idx_v, rows_v, sem):
        wid = lax.axis_index("s") * 2 + lax.axis_index("c")
        for j in range(_NCH):
            base = wid * (_CH * _NCH) + j * _CH
            pltpu.sync_copy(pos_hbm.at[wid, j], idx_v)
            pltpu.async_copy(h_hbm.at[idx_v], rows_v, sem).wait()
            pltpu.sync_copy(rows_v, out_hbm.at[pl.ds(base, _CH)])

    return k(hs, pos3)


def _moe_body(sched_ref, valid_ref, xs_ref, g_ref, u_ref, d_ref, o_ref):
    i = pl.program_id(0)

    @pl.when(valid_ref[i] == 1)
    def _():
        xb = xs_ref[...].astype(bf16)
        g = _dot_nt(xb, g_ref[0])
        u = _dot_nt(xb, u_ref[0])
        act = (g * jax.nn.sigmoid(g) * u).astype(bf16)
        o_ref[...] = _dot_nt(act, d_ref[0])


def _moe_grouped(sched1d, valid1d, xs, gbf, ubf, dbf):
    grid_spec = pltpu.PrefetchScalarGridSpec(
        num_scalar_prefetch=2,
        grid=(NBLK,),
        in_specs=[
            pl.BlockSpec((G, D), lambda i, sr, vr: (i, 0)),
            pl.BlockSpec((1, M, D), lambda i, sr, vr: (sr[i], 0, 0)),
            pl.BlockSpec((1, M, D), lambda i, sr, vr: (sr[i], 0, 0)),
            pl.BlockSpec((1, D, M), lambda i, sr, vr: (sr[i], 0, 0)),
        ],
        out_specs=pl.BlockSpec((G, D), lambda i, sr, vr: (i, 0)),
    )
    return pl.pallas_call(
        _moe_body,
        grid_spec=grid_spec,
        out_shape=jax.ShapeDtypeStruct((NPAD, D), f32),
    )(sched1d, valid1d, xs, gbf, ubf, dbf)


def _shared_body(x_ref, g_ref, u_ref, d_ref, o_ref):
    xb = x_ref[...].astype(bf16)
    g = _dot_nt(xb, g_ref[...])
    u = _dot_nt(xb, u_ref[...])
    act = (g * jax.nn.sigmoid(g) * u).astype(bf16)
    o_ref[...] = _dot_nt(act, d_ref[...])


def _shared_expert(x1, shgbf, shubf, shdbf):
    return pl.pallas_call(
        _shared_body,
        grid=(4,),
        in_specs=[
            pl.BlockSpec((512, D), lambda r: (r, 0)),
            pl.BlockSpec((M, D), lambda r: (0, 0)),
            pl.BlockSpec((M, D), lambda r: (0, 0)),
            pl.BlockSpec((D, M), lambda r: (0, 0)),
        ],
        out_specs=pl.BlockSpec((512, D), lambda r: (r, 0)),
        out_shape=jax.ShapeDtypeStruct((T, D), f32),
    )(x1, shgbf, shubf, shdbf)


def _combine_body(h0_ref, h1_ref, wp_ref, sh_ref, x1_ref, lw_ref, lb_ref,
                  o_ref):
    w0 = wp_ref[:, 0:1]
    w1 = wp_ref[:, 1:2]
    y = (x1_ref[...] + sh_ref[...] + w0 * h0_ref[...].astype(f32)
         + w1 * h1_ref[...].astype(f32))
    mu = jnp.mean(y, axis=1, keepdims=True)
    var = jnp.mean((y - mu) ** 2, axis=1, keepdims=True)
    o_ref[...] = (y - mu) / jnp.sqrt(var + 1e-5) * lw_ref[...] + lb_ref[...]


def _combine(h01, wpair, shared, x1, ln2w2d, ln2b2d):
    return pl.pallas_call(
        _combine_body,
        grid=(4,),
        in_specs=[
            pl.BlockSpec((512, D), lambda r: (r, 0)),
            pl.BlockSpec((512, D), lambda r: (4 + r, 0)),
            pl.BlockSpec((512, 128), lambda r: (r, 0)),
            pl.BlockSpec((512, D), lambda r: (r, 0)),
            pl.BlockSpec((512, D), lambda r: (r, 0)),
            pl.BlockSpec((1, D), lambda r: (0, 0)),
            pl.BlockSpec((1, D), lambda r: (0, 0)),
        ],
        out_specs=pl.BlockSpec((512, D), lambda r: (r, 0)),
        out_shape=jax.ShapeDtypeStruct((T, D), f32),
    )(h01, h01, wpair, shared, x1, ln2w2d, ln2b2d)


def kernel(x, in_proj_w, in_proj_b, out_proj_w, out_proj_b, ln1_w, ln1_b,
           ln2_w, ln2_b, router_w, e_bias, gate_w, up_w, down_w, sh_gate_w,
           sh_up_w, sh_down_w):
    x2d = x.reshape(T, D)
    xbf = x2d.astype(bf16)

    qkv = _qkv_proj(xbf, in_proj_w.astype(bf16), in_proj_b.reshape(1, 3 * D))
    ao = _attention(qkv)

    rwpad = jnp.zeros((128, D), bf16).at[:E].set(router_w.astype(bf16))
    x1, logits = _postattn(ao, out_proj_w.astype(bf16),
                           out_proj_b.reshape(1, D), x2d,
                           ln1_w.reshape(1, D), ln1_b.reshape(1, D), rwpad)

    ebpad = jnp.zeros((1, 128), f32).at[0, :E].set(e_bias)
    wpair, pos, sched, valid = _route(logits, ebpad)
    pos3 = pos.reshape(_NW, _NCH, _CH)
    sched1d = sched[:NBLK, 0]
    valid1d = valid[:NBLK, 0]

    xs = _sc_scatter_rows(x1, pos3)
    shared = _shared_expert(x1, sh_gate_w.astype(bf16), sh_up_w.astype(bf16),
                            sh_down_w.astype(bf16))

    hs = _moe_grouped(sched1d, valid1d, xs, gate_w.astype(bf16),
                      up_w.astype(bf16), down_w.astype(bf16))
    h01 = _sc_gather_rows(hs, pos3)

    out = _combine(h01, wpair, shared, x1, ln2_w.reshape(1, D),
                   ln2_b.reshape(1, D))
    return out.reshape(1, T, D)
